# baseline (speedup 1.0000x reference)
import functools

import jax
import jax.numpy as jnp
from jax import lax
from jax.experimental import pallas as pl
from jax.experimental.pallas import tpu as pltpu
from jax.experimental.pallas import tpu_sc as plsc

DIM_E = 64
BATCH = 1024
B_TOTAL = 2 * BATCH

_SC_INFO = plsc.get_sparse_core_info()
_NC = _SC_INFO.num_cores
_NS = _SC_INFO.num_subcores
_NW = _NC * _NS
_B_PER_W = B_TOTAL // _NW


def _gather_body(table_hbm, idx_hbm, out_hbm, idx_v, rows_v, sem):
    wid = lax.axis_index("s") * _NC + lax.axis_index("c")
    base = wid * _B_PER_W
    pltpu.sync_copy(idx_hbm.at[pl.ds(base, _B_PER_W)], idx_v)
    pltpu.async_copy(table_hbm.at[idx_v], rows_v, sem).wait()
    pltpu.sync_copy(rows_v, out_hbm.at[pl.ds(base, _B_PER_W)])


@jax.jit
def _sc_gather(id_embedding, idx):
    mesh = plsc.VectorSubcoreMesh(core_axis_name="c", subcore_axis_name="s")
    return pl.kernel(
        _gather_body,
        mesh=mesh,
        out_type=jax.ShapeDtypeStruct((B_TOTAL, DIM_E), jnp.float32),
        scratch_types=[
            pltpu.VMEM((_B_PER_W,), jnp.int32),
            pltpu.VMEM((_B_PER_W, DIM_E), jnp.float32),
            pltpu.SemaphoreType.DMA,
        ],
        compiler_params=pltpu.CompilerParams(use_tc_tiling_on_sc=False),
    )(id_embedding, idx)


def _mm_body(rows_ref, o_ref):
    u = rows_ref[0:BATCH, :]
    v = rows_ref[BATCH:B_TOTAL, :]
    o_ref[...] = lax.dot_general(
        u, v, (((1,), (1,)), ((), ())), preferred_element_type=jnp.float32
    )


@jax.jit
def _tc_matmul(rows):
    return pl.pallas_call(
        _mm_body,
        out_shape=jax.ShapeDtypeStruct((BATCH, BATCH), jnp.float32),
    )(rows)


def kernel(user_tensor, item_tensor, id_embedding):
    idx = jnp.concatenate([user_tensor, item_tensor]).astype(jnp.int32)
    rows = _sc_gather(id_embedding, idx)
    return _tc_matmul(rows)


# speedup vs baseline: 1.7155x; 1.7155x over previous
import jax
import jax.numpy as jnp
from jax import lax
from jax.experimental import pallas as pl
from jax.experimental.pallas import tpu as pltpu
from jax.experimental.pallas import tpu_sc as plsc

DIM_E = 64
BATCH = 1024
B_TOTAL = 2 * BATCH

_SC_INFO = plsc.get_sparse_core_info()
_NC = _SC_INFO.num_cores
_NS = _SC_INFO.num_subcores
_NW = _NC * _NS
_B_PER_W = B_TOTAL // _NW


def _gather_body(table_hbm, idx_hbm, out_hbm, idx_v, rows_v, sem):
    wid = lax.axis_index("s") * _NC + lax.axis_index("c")
    base = wid * _B_PER_W
    pltpu.sync_copy(idx_hbm.at[pl.ds(base, _B_PER_W)], idx_v)
    copies = []
    for g in range(_B_PER_W // 16):
        vec = idx_v[pl.ds(g * 16, 16)]
        for l in range(16):
            k = g * 16 + l
            j = vec[l]
            copies.append(
                pltpu.async_copy(
                    table_hbm.at[pl.ds(j, 1)], rows_v.at[pl.ds(k, 1)], sem
                )
            )
    for c in copies:
        c.wait()
    pltpu.sync_copy(rows_v, out_hbm.at[pl.ds(base, _B_PER_W)])


def _sc_gather(table, idx):
    mesh = plsc.VectorSubcoreMesh(core_axis_name="c", subcore_axis_name="s")
    return pl.kernel(
        _gather_body,
        mesh=mesh,
        out_type=jax.ShapeDtypeStruct((B_TOTAL, DIM_E), jnp.float32),
        scratch_types=[
            pltpu.VMEM((_B_PER_W,), jnp.int32),
            pltpu.VMEM((_B_PER_W, DIM_E), jnp.float32),
            pltpu.SemaphoreType.DMA,
        ],
    )(table, idx)


def _mm_body(rows_ref, o_ref):
    u = rows_ref[0:BATCH, :]
    v = rows_ref[BATCH:B_TOTAL, :]
    o_ref[...] = lax.dot_general(
        u, v, (((1,), (1,)), ((), ())), preferred_element_type=jnp.float32
    )


def _tc_matmul(rows):
    return pl.pallas_call(
        _mm_body,
        out_shape=jax.ShapeDtypeStruct((BATCH, BATCH), jnp.float32),
    )(rows)


def kernel(user_tensor, item_tensor, id_embedding):
    idx = jnp.concatenate([user_tensor, item_tensor]).astype(jnp.int32)
    rows = _sc_gather(id_embedding, idx)
    return _tc_matmul(rows)


# speedup vs baseline: 11.1451x; 6.4966x over previous
import jax
import jax.numpy as jnp
from jax import lax
from jax.experimental import pallas as pl
from jax.experimental.pallas import tpu as pltpu
from jax.experimental.pallas import tpu_sc as plsc

DIM_E = 64
BATCH = 1024
B_TOTAL = 2 * BATCH
LANES = 128

_SC_INFO = plsc.get_sparse_core_info()
_NC = _SC_INFO.num_cores
_NS = _SC_INFO.num_subcores
_NW = _NC * _NS
_B_PER_W = B_TOTAL // _NW
_WAVE = 4
_NWAVES = _B_PER_W // _WAVE


def _gather_body(table_hbm, idx_hbm, out_hbm, idx_v, block_v, rows_v, sem0, sem1):
    wid = lax.axis_index("s") * _NC + lax.axis_index("c")
    base = wid * _B_PER_W
    pltpu.sync_copy(idx_hbm.at[pl.ds(base, _B_PER_W)], idx_v)
    vecs = [idx_v[pl.ds(g * 16, 16)] for g in range(_B_PER_W // 16)]
    sems = [sem0, sem1]

    def fire(w):
        descs = []
        for i in range(_WAVE):
            k = w * _WAVE + i
            j = vecs[k // 16][k % 16]
            jb = pl.multiple_of((j >> 7) << 7, LANES)
            descs.append(
                pltpu.async_copy(
                    table_hbm.at[:, pl.ds(jb, LANES)],
                    block_v.at[(w % 2) * _WAVE + i],
                    sems[w % 2],
                )
            )
        return descs

    def select(w, descs):
        for d in descs:
            d.wait()
        for i in range(_WAVE):
            k = w * _WAVE + i
            j = vecs[k // 16][k % 16]
            cvec = jnp.full((16,), j & (LANES - 1), dtype=jnp.int32)
            kvec = jnp.full((16,), k, dtype=jnp.int32)
            blk = block_v.at[(w % 2) * _WAVE + i]
            for g in range(DIM_E // 16):
                ridx = lax.iota(jnp.int32, 16) + g * 16
                vals = plsc.load_gather(blk, [ridx, cvec])
                plsc.store_scatter(rows_v, [kvec, ridx], vals)

    descs = fire(0)
    for w in range(_NWAVES):
        nxt = fire(w + 1) if w + 1 < _NWAVES else []
        select(w, descs)
        descs = nxt
    pltpu.sync_copy(rows_v, out_hbm.at[pl.ds(base, _B_PER_W)])


def _sc_gather(table_t, idx):
    mesh = plsc.VectorSubcoreMesh(core_axis_name="c", subcore_axis_name="s")
    return pl.kernel(
        _gather_body,
        mesh=mesh,
        out_type=jax.ShapeDtypeStruct((B_TOTAL, DIM_E), jnp.float32),
        scratch_types=[
            pltpu.VMEM((_B_PER_W,), jnp.int32),
            pltpu.VMEM((2 * _WAVE, DIM_E, LANES), jnp.float32),
            pltpu.VMEM((_B_PER_W, DIM_E), jnp.float32),
            pltpu.SemaphoreType.DMA,
            pltpu.SemaphoreType.DMA,
        ],
        compiler_params=pltpu.CompilerParams(needs_layout_passes=False),
    )(table_t, idx)


def _mm_body(rows_ref, o_ref):
    u = rows_ref[0:BATCH, :]
    v = rows_ref[BATCH:B_TOTAL, :]
    o_ref[...] = lax.dot_general(
        u, v, (((1,), (1,)), ((), ())), preferred_element_type=jnp.float32
    )


def _tc_matmul(rows):
    return pl.pallas_call(
        _mm_body,
        out_shape=jax.ShapeDtypeStruct((BATCH, BATCH), jnp.float32),
    )(rows)


def kernel(user_tensor, item_tensor, id_embedding):
    idx = jnp.concatenate([user_tensor, item_tensor]).astype(jnp.int32)
    rows = _sc_gather(id_embedding.T, idx)
    return _tc_matmul(rows)


# speedup vs baseline: 11.5185x; 1.0335x over previous
import jax
import jax.numpy as jnp
from jax import lax
from jax.experimental import pallas as pl
from jax.experimental.pallas import tpu as pltpu
from jax.experimental.pallas import tpu_sc as plsc

DIM_E = 64
BATCH = 1024
B_TOTAL = 2 * BATCH
LANES = 128

_SC_INFO = plsc.get_sparse_core_info()
_NC = _SC_INFO.num_cores
_NS = _SC_INFO.num_subcores
_NW = _NC * _NS
_B_PER_W = B_TOTAL // _NW
_H_PER_W = _B_PER_W // 2
_WAVE = 4
_DEPTH = 3
_NWAVES = _B_PER_W // _WAVE


def _gather_body(table_hbm, user_hbm, item_hbm, out_hbm,
                 idx_v, block_v, rows_v, sem0, sem1, sem2):
    wid = lax.axis_index("s") * _NC + lax.axis_index("c")
    ubase = wid * _H_PER_W
    pltpu.sync_copy(user_hbm.at[pl.ds(ubase, _H_PER_W)],
                    idx_v.at[pl.ds(0, _H_PER_W)])
    pltpu.sync_copy(item_hbm.at[pl.ds(ubase, _H_PER_W)],
                    idx_v.at[pl.ds(_H_PER_W, _H_PER_W)])
    vecs = [idx_v[pl.ds(g * 16, 16)] for g in range(_B_PER_W // 16)]
    sems = [sem0, sem1, sem2]

    def fire(w):
        descs = []
        for i in range(_WAVE):
            k = w * _WAVE + i
            j = vecs[k // 16][k % 16]
            jb = pl.multiple_of((j >> 7) << 7, LANES)
            descs.append(
                pltpu.async_copy(
                    table_hbm.at[:, pl.ds(jb, LANES)],
                    block_v.at[(w % _DEPTH) * _WAVE + i],
                    sems[w % _DEPTH],
                )
            )
        return descs

    def select(w, descs):
        for d in descs:
            d.wait()
        for i in range(_WAVE):
            k = w * _WAVE + i
            j = vecs[k // 16][k % 16]
            cvec = jnp.full((16,), j & (LANES - 1), dtype=jnp.int32)
            kvec = jnp.full((16,), k, dtype=jnp.int32)
            blk = block_v.at[(w % _DEPTH) * _WAVE + i]
            for g in range(DIM_E // 16):
                ridx = lax.iota(jnp.int32, 16) + g * 16
                vals = plsc.load_gather(blk, [ridx, cvec])
                plsc.store_scatter(rows_v, [kvec, ridx], vals)

    pending = {0: fire(0), 1: fire(1)}
    for w in range(_NWAVES):
        if w + 2 < _NWAVES:
            pending[w + 2] = fire(w + 2)
        select(w, pending.pop(w))
    pltpu.sync_copy(rows_v.at[pl.ds(0, _H_PER_W)],
                    out_hbm.at[pl.ds(ubase, _H_PER_W)])
    pltpu.sync_copy(rows_v.at[pl.ds(_H_PER_W, _H_PER_W)],
                    out_hbm.at[pl.ds(BATCH + ubase, _H_PER_W)])


def _sc_gather(table_t, user_idx, item_idx):
    mesh = plsc.VectorSubcoreMesh(core_axis_name="c", subcore_axis_name="s")
    return pl.kernel(
        _gather_body,
        mesh=mesh,
        out_type=jax.ShapeDtypeStruct((B_TOTAL, DIM_E), jnp.float32),
        scratch_types=[
            pltpu.VMEM((_B_PER_W,), jnp.int32),
            pltpu.VMEM((_DEPTH * _WAVE, DIM_E, LANES), jnp.float32),
            pltpu.VMEM((_B_PER_W, DIM_E), jnp.float32),
            pltpu.SemaphoreType.DMA,
            pltpu.SemaphoreType.DMA,
            pltpu.SemaphoreType.DMA,
        ],
        compiler_params=pltpu.CompilerParams(needs_layout_passes=False),
    )(table_t, user_idx, item_idx)


def _mm_body(rows_ref, o_ref):
    u = rows_ref[0:BATCH, :]
    v = rows_ref[BATCH:B_TOTAL, :]
    o_ref[...] = lax.dot_general(
        u, v, (((1,), (1,)), ((), ())), preferred_element_type=jnp.float32
    )


def _tc_matmul(rows):
    return pl.pallas_call(
        _mm_body,
        out_shape=jax.ShapeDtypeStruct((BATCH, BATCH), jnp.float32),
    )(rows)


def kernel(user_tensor, item_tensor, id_embedding):
    rows = _sc_gather(
        id_embedding.T,
        user_tensor.astype(jnp.int32),
        item_tensor.astype(jnp.int32),
    )
    return _tc_matmul(rows)
